# Initial kernel scaffold; baseline (speedup 1.0000x reference)
import functools

import jax
import jax.numpy as jnp
from jax import lax
from jax.experimental import pallas as pl
from jax.experimental.pallas import tpu as pltpu
from jax.experimental.pallas import tpu_sc as plsc

N = 10000
D = 128
E = 320000
NC = 2
NS = 16
CHUNK = 128
CPW = 79
EPW = CPW * CHUNK
EPAD = NC * NS * EPW
NPAD = 10240
RPW = NPAD // NS


def _sc_agg(x, src, dst, with_cnt):
    mesh = plsc.VectorSubcoreMesh(core_axis_name="c", subcore_axis_name="s")
    out_type = [jax.ShapeDtypeStruct((NC, NPAD, D), jnp.float32)]
    scratch = [
        pltpu.VMEM((CHUNK,), jnp.int32),
        pltpu.VMEM((CHUNK,), jnp.int32),
        pltpu.VMEM((CHUNK, D), jnp.float32),
        pltpu.VMEM((CHUNK, D), jnp.float32),
        pltpu.VMEM_SHARED((NPAD, D), jnp.float32),
        pltpu.SemaphoreType.DMA,
    ]
    if with_cnt:
        out_type.append(jax.ShapeDtypeStruct((NC, NPAD, 16), jnp.float32))
        scratch.insert(4, pltpu.VMEM((CHUNK, 16), jnp.float32))
        scratch.insert(5, pltpu.VMEM((CHUNK, 16), jnp.float32))
        scratch.append(pltpu.VMEM_SHARED((NPAD, 16), jnp.float32))

    def body(x_hbm, src_hbm, dst_hbm, *refs):
        if with_cnt:
            (p_hbm, cnt_hbm, srcv, dstv, rows, zbuf, ones16, z16, agg, sem,
             cnts) = refs
        else:
            p_hbm, srcv, dstv, rows, zbuf, agg, sem = refs
        c = lax.axis_index("c")
        s = lax.axis_index("s")

        zero = jnp.zeros((16,), jnp.float32)

        @pl.loop(0, CHUNK)
        def _(i):
            for j in range(D // 16):
                zbuf[i, pl.ds(j * 16, 16)] = zero
            if with_cnt:
                ones16[i, pl.ds(0, 16)] = zero + 1.0
                z16[i, pl.ds(0, 16)] = zero

        for k in range(RPW // CHUNK):
            r0 = s * RPW + k * CHUNK
            pltpu.sync_copy(zbuf, agg.at[pl.ds(r0, CHUNK)])
            if with_cnt:
                pltpu.sync_copy(z16, cnts.at[pl.ds(r0, CHUNK)])
        plsc.subcore_barrier()

        base = (c * NS + s) * EPW

        @pl.loop(0, CPW)
        def _(j):
            off = base + j * CHUNK
            pltpu.sync_copy(src_hbm.at[pl.ds(off, CHUNK)], srcv)
            pltpu.sync_copy(dst_hbm.at[pl.ds(off, CHUNK)], dstv)
            pltpu.async_copy(x_hbm.at[srcv], rows, sem).wait()
            pltpu.sync_copy(rows, agg.at[dstv], add=True)
            if with_cnt:
                pltpu.sync_copy(ones16, cnts.at[dstv], add=True)

        plsc.subcore_barrier()

        r0 = s * RPW
        pltpu.sync_copy(agg.at[pl.ds(r0, RPW)], p_hbm.at[c].at[pl.ds(r0, RPW)])
        if with_cnt:
            pltpu.sync_copy(cnts.at[pl.ds(r0, RPW)],
                            cnt_hbm.at[c].at[pl.ds(r0, RPW)])

    k = pl.kernel(body, mesh=mesh, out_type=tuple(out_type),
                  scratch_types=scratch)
    return k(x, src, dst)


def _tc_layer(p, cnt, x, WlT, bl, WrT, relu):
    def body(p0_ref, p1_ref, c0_ref, c1_ref, x_ref, wl_ref, bl_ref, wr_ref,
             o_ref):
        cntv = c0_ref[:, :1] + c1_ref[:, :1]
        mean = (p0_ref[...] + p1_ref[...]) / jnp.maximum(cntv, 1.0)
        acc = jnp.dot(mean, wl_ref[...], precision=lax.Precision.HIGHEST,
                      preferred_element_type=jnp.float32)
        acc += jnp.dot(x_ref[...], wr_ref[...],
                       precision=lax.Precision.HIGHEST,
                       preferred_element_type=jnp.float32)
        acc += bl_ref[...]
        o_ref[...] = jnp.maximum(acc, 0.0) if relu else acc

    return pl.pallas_call(
        body,
        out_shape=jax.ShapeDtypeStruct((N, D), jnp.float32),
    )(p[0, :N], p[1, :N], cnt[0, :N], cnt[1, :N], x, WlT, bl.reshape(1, D),
      WrT)


def kernel(x, edge_index, W1l, b1l, W1r, W2l, b2l, W2r):
    src = edge_index[0].astype(jnp.int32)
    dst = edge_index[1].astype(jnp.int32)
    pad = EPAD - E
    src_p = jnp.concatenate([src, jnp.zeros((pad,), jnp.int32)])
    dst_p = jnp.concatenate([dst, jnp.full((pad,), N, jnp.int32)])

    p1, cnt = _sc_agg(x, src_p, dst_p, with_cnt=True)
    h = _tc_layer(p1, cnt, x, W1l.T, b1l, W1r.T, relu=True)
    (p2,) = _sc_agg(h, src_p, dst_p, with_cnt=False)
    return _tc_layer(p2, cnt, h, W2l.T, b2l, W2r.T, relu=False)


# speedup vs baseline: 3.6519x; 3.6519x over previous
import dataclasses

import jax
import jax.numpy as jnp
from jax import lax
from jax.experimental import pallas as pl
from jax.experimental.pallas import tpu as pltpu
from jax.experimental.pallas import tpu_sc as plsc

N = 10000
D = 128
E = 320000
NC = 2
NS = 16
L = 16
CHUNK = 64
NCH = 158
EPW = NCH * CHUNK
EPAD = NC * NS * EPW
NPAD = 10240
RPW = NPAD // NS


def _sc_agg(x, src, dst, with_cnt):
    mesh = plsc.VectorSubcoreMesh(core_axis_name="c", subcore_axis_name="s")
    out_type = [jax.ShapeDtypeStruct((NC * NPAD, D), jnp.float32)]
    scratch = [
        pltpu.VMEM((CHUNK,), jnp.int32),
        pltpu.VMEM((CHUNK,), jnp.int32),
        pltpu.VMEM((CHUNK, D), jnp.float32),
        pltpu.VMEM_SHARED((NPAD, D), jnp.float32),
        pltpu.SemaphoreType.DMA,
    ]
    if with_cnt:
        out_type.append(
            jax.ShapeDtypeStruct((NC * NPAD // 128, 128), jnp.float32))
        scratch.append(pltpu.VMEM((NPAD,), jnp.float32))
        scratch.append(pltpu.VMEM((NS, 128), jnp.float32))
        scratch.append(pltpu.VMEM((8, 128), jnp.float32))
        scratch.append(pltpu.VMEM_SHARED((NS, NPAD), jnp.float32))

    def body(x_hbm, src_hbm, dst_hbm, *refs):
        if with_cnt:
            (p_hbm, cnt_hbm, srcv, dstv, rows, agg, sem,
             cntloc, redbuf, cpk, stage) = refs
        else:
            p_hbm, srcv, dstv, rows, agg, sem = refs
        c = lax.axis_index("c")
        s = lax.axis_index("s")

        zero = jnp.zeros((16,), jnp.float32)

        @pl.loop(0, CHUNK)
        def _(i):
            for j in range(D // 16):
                rows[i, pl.ds(j * 16, 16)] = zero

        if with_cnt:
            @pl.loop(0, NPAD // L)
            def _(i):
                cntloc[pl.ds(i * L, L)] = jnp.zeros((L,), jnp.float32)

        for k in range(RPW // CHUNK):
            r0 = s * RPW + k * CHUNK
            pltpu.sync_copy(rows, agg.at[pl.ds(r0, CHUNK)])
        plsc.subcore_barrier()

        base = (c * NS + s) * EPW
        ones = jnp.zeros((L,), jnp.float32) + 1.0

        @pl.loop(0, NCH)
        def _(j):
            off = base + j * CHUNK
            pltpu.sync_copy(src_hbm.at[pl.ds(off, CHUNK)], srcv)
            pltpu.sync_copy(dst_hbm.at[pl.ds(off, CHUNK)], dstv)
            pltpu.async_copy(x_hbm.at[srcv], rows, sem).wait()
            pltpu.sync_copy(rows, agg.at[dstv], add=True)
            if with_cnt:
                for k in range(CHUNK // L):
                    plsc.addupdate_scatter(
                        cntloc, [dstv[pl.ds(k * L, L)]], ones)

        plsc.subcore_barrier()

        r0 = s * RPW
        pltpu.sync_copy(agg.at[pl.ds(r0, RPW)],
                        p_hbm.at[pl.ds(c * NPAD + r0, RPW)])

        if with_cnt:
            pltpu.sync_copy(cntloc, stage.at[s])
            plsc.subcore_barrier()

            @pl.when(s < NPAD // 1024)
            def _():
                for r in range(8):
                    pltpu.sync_copy(
                        stage.at[:, pl.ds(s * 1024 + r * 128, 128)], redbuf)

                    for g in range(128 // L):
                        acc = redbuf[0, pl.ds(g * L, L)]
                        for t in range(1, NS):
                            acc += redbuf[t, pl.ds(g * L, L)]
                        cpk[r, pl.ds(g * L, L)] = acc

                pltpu.sync_copy(
                    cpk, cnt_hbm.at[pl.ds(c * (NPAD // 128) + s * 8, 8)])

    cp = pltpu.CompilerParams()
    if "needs_layout_passes" in pltpu.CompilerParams.__dataclass_fields__:
        cp = dataclasses.replace(cp, needs_layout_passes=False)
    k = pl.kernel(body, mesh=mesh, out_type=tuple(out_type),
                  scratch_types=scratch, compiler_params=cp)
    return k(x, src, dst)


def _tc_layer(p, cnt, x, WlT, bl, WrT, relu):
    def body(p0_ref, p1_ref, c0_ref, c1_ref, x_ref, wl_ref, bl_ref, wr_ref,
             o_ref):
        cntv = c0_ref[...] + c1_ref[...]
        mean = (p0_ref[...] + p1_ref[...]) / jnp.maximum(cntv, 1.0)
        acc = jnp.dot(mean, wl_ref[...], precision=lax.Precision.HIGHEST,
                      preferred_element_type=jnp.float32)
        acc += jnp.dot(x_ref[...], wr_ref[...],
                       precision=lax.Precision.HIGHEST,
                       preferred_element_type=jnp.float32)
        acc += bl_ref[...]
        o_ref[...] = jnp.maximum(acc, 0.0) if relu else acc

    R = 2000
    row = pl.BlockSpec((R, D), lambda i: (i, 0))
    row1 = pl.BlockSpec((R, 1), lambda i: (i, 0))
    full = pl.BlockSpec((D, D), lambda i: (0, 0))
    return pl.pallas_call(
        body,
        grid=(N // R,),
        in_specs=[row, row, row1, row1, row, full,
                  pl.BlockSpec((1, D), lambda i: (0, 0)), full],
        out_specs=row,
        out_shape=jax.ShapeDtypeStruct((N, D), jnp.float32),
    )(p[:N], p[NPAD:NPAD + N], cnt[0], cnt[1], x, WlT,
      bl.reshape(1, D), WrT)


def kernel(x, edge_index, W1l, b1l, W1r, W2l, b2l, W2r):
    src = edge_index[0].astype(jnp.int32)
    dst = edge_index[1].astype(jnp.int32)
    pad = EPAD - E
    src_p = jnp.concatenate([src, jnp.zeros((pad,), jnp.int32)])
    dst_p = jnp.concatenate([dst, jnp.full((pad,), N, jnp.int32)])

    p1, cnt_pk = _sc_agg(x, src_p, dst_p, with_cnt=True)
    cnt = cnt_pk.reshape(NC, NPAD)[:, :N, None]
    h = _tc_layer(p1, cnt, x, W1l.T, b1l, W1r.T, relu=True)
    (p2,) = _sc_agg(h, src_p, dst_p, with_cnt=False)
    return _tc_layer(p2, cnt, h, W2l.T, b2l, W2r.T, relu=False)


# speedup vs baseline: 3.7336x; 1.0224x over previous
import dataclasses

import jax
import jax.numpy as jnp
from jax import lax
from jax.experimental import pallas as pl
from jax.experimental.pallas import tpu as pltpu
from jax.experimental.pallas import tpu_sc as plsc

N = 10000
D = 128
E = 320000
NC = 2
NS = 16
L = 16
CHUNK = 64
BLK = 8
NBLK = 20
NCH = BLK * NBLK
EPW = NCH * CHUNK
EPAD = NC * NS * EPW
NPAD = 10240
RPW = NPAD // NS


def _compiler_params():
    cp = pltpu.CompilerParams()
    if "needs_layout_passes" in pltpu.CompilerParams.__dataclass_fields__:
        cp = dataclasses.replace(cp, needs_layout_passes=False)
    return cp


def _sc_agg(x, src2, dst2):
    mesh = plsc.VectorSubcoreMesh(core_axis_name="c", subcore_axis_name="s")

    def body(x_hbm, src_hbm, dst_hbm, p_hbm, src8, dst8, rows0, rows1, agg,
             sem0, sem1):
        c = lax.axis_index("c")
        s = lax.axis_index("s")

        zero = jnp.zeros((16,), jnp.float32)

        @pl.loop(0, CHUNK)
        def _(i):
            for j in range(D // 16):
                rows0[i, pl.ds(j * 16, 16)] = zero

        for k in range(RPW // CHUNK):
            r0 = s * RPW + k * CHUNK
            pltpu.sync_copy(rows0, agg.at[pl.ds(r0, CHUNK)])
        plsc.subcore_barrier()

        wrow = (c * NS + s) * NCH
        rows = (rows0, rows1)
        sems = (sem0, sem1)

        @pl.loop(0, NBLK)
        def _(b):
            blk = wrow + b * BLK
            pltpu.sync_copy(src_hbm.at[pl.ds(blk, BLK)], src8)
            pltpu.sync_copy(dst_hbm.at[pl.ds(blk, BLK)], dst8)
            pltpu.async_copy(x_hbm.at[src8.at[0]], rows0, sem0)
            for j in range(BLK):
                if j + 1 < BLK:
                    pltpu.async_copy(x_hbm.at[src8.at[j + 1]],
                                     rows[(j + 1) % 2], sems[(j + 1) % 2])
                pltpu.make_async_copy(x_hbm.at[src8.at[j]], rows[j % 2],
                                      sems[j % 2]).wait()
                pltpu.sync_copy(rows[j % 2], agg.at[dst8.at[j]], add=True)

        plsc.subcore_barrier()

        r0 = s * RPW
        pltpu.sync_copy(agg.at[pl.ds(r0, RPW)],
                        p_hbm.at[pl.ds(c * NPAD + r0, RPW)])

    k = pl.kernel(
        body, mesh=mesh,
        out_type=jax.ShapeDtypeStruct((NC * NPAD, D), jnp.float32),
        scratch_types=[
            pltpu.VMEM((BLK, CHUNK), jnp.int32),
            pltpu.VMEM((BLK, CHUNK), jnp.int32),
            pltpu.VMEM((CHUNK, D), jnp.float32),
            pltpu.VMEM((CHUNK, D), jnp.float32),
            pltpu.VMEM_SHARED((NPAD, D), jnp.float32),
            pltpu.SemaphoreType.DMA,
            pltpu.SemaphoreType.DMA,
        ],
        compiler_params=_compiler_params())
    return k(x, src2, dst2)


def _sc_count(dst2):
    mesh = plsc.VectorSubcoreMesh(core_axis_name="c", subcore_axis_name="s")

    def body(dst_hbm, cnt_hbm, dst8, cntloc, redbuf, cpk, stage):
        c = lax.axis_index("c")
        s = lax.axis_index("s")

        @pl.loop(0, NPAD // L)
        def _(i):
            cntloc[pl.ds(i * L, L)] = jnp.zeros((L,), jnp.float32)

        wrow = (c * NS + s) * NCH
        ones = jnp.zeros((L,), jnp.float32) + 1.0

        @pl.loop(0, NBLK)
        def _(b):
            pltpu.sync_copy(dst_hbm.at[pl.ds(wrow + b * BLK, BLK)], dst8)
            for j in range(BLK):
                for k in range(CHUNK // L):
                    plsc.addupdate_scatter(
                        cntloc, [dst8[j, pl.ds(k * L, L)]], ones)

        pltpu.sync_copy(cntloc, stage.at[s])
        plsc.subcore_barrier()

        @pl.when(s < NPAD // 1024)
        def _():
            for r in range(8):
                pltpu.sync_copy(
                    stage.at[:, pl.ds(s * 1024 + r * 128, 128)], redbuf)

                for g in range(128 // L):
                    acc = redbuf[0, pl.ds(g * L, L)]
                    for t in range(1, NS):
                        acc += redbuf[t, pl.ds(g * L, L)]
                    cpk[r, pl.ds(g * L, L)] = acc

            pltpu.sync_copy(
                cpk, cnt_hbm.at[pl.ds(c * (NPAD // 128) + s * 8, 8)])

    k = pl.kernel(
        body, mesh=mesh,
        out_type=jax.ShapeDtypeStruct((NC * NPAD // 128, 128), jnp.float32),
        scratch_types=[
            pltpu.VMEM((BLK, CHUNK), jnp.int32),
            pltpu.VMEM((NPAD,), jnp.float32),
            pltpu.VMEM((NS, 128), jnp.float32),
            pltpu.VMEM((8, 128), jnp.float32),
            pltpu.VMEM_SHARED((NS, NPAD), jnp.float32),
        ],
        compiler_params=_compiler_params())
    return k(dst2)


def _tc_layer(p, cnt, x, WlT, bl, WrT, relu):
    def body(p0_ref, p1_ref, c0_ref, c1_ref, x_ref, wl_ref, bl_ref, wr_ref,
             o_ref):
        cntv = c0_ref[...] + c1_ref[...]
        mean = (p0_ref[...] + p1_ref[...]) / jnp.maximum(cntv, 1.0)
        acc = jnp.dot(mean, wl_ref[...], precision=lax.Precision.HIGHEST,
                      preferred_element_type=jnp.float32)
        acc += jnp.dot(x_ref[...], wr_ref[...],
                       precision=lax.Precision.HIGHEST,
                       preferred_element_type=jnp.float32)
        acc += bl_ref[...]
        o_ref[...] = jnp.maximum(acc, 0.0) if relu else acc

    R = 2000
    row = pl.BlockSpec((R, D), lambda i: (i, 0))
    row1 = pl.BlockSpec((R, 1), lambda i: (i, 0))
    full = pl.BlockSpec((D, D), lambda i: (0, 0))
    return pl.pallas_call(
        body,
        grid=(N // R,),
        in_specs=[row, row, row1, row1, row, full,
                  pl.BlockSpec((1, D), lambda i: (0, 0)), full],
        out_specs=row,
        out_shape=jax.ShapeDtypeStruct((N, D), jnp.float32),
    )(p[:N], p[NPAD:NPAD + N], cnt[0], cnt[1], x, WlT,
      bl.reshape(1, D), WrT)


def kernel(x, edge_index, W1l, b1l, W1r, W2l, b2l, W2r):
    src = edge_index[0].astype(jnp.int32)
    dst = edge_index[1].astype(jnp.int32)
    pad = EPAD - E
    src2 = jnp.concatenate([src, jnp.zeros((pad,), jnp.int32)])
    dst2 = jnp.concatenate([dst, jnp.full((pad,), N, jnp.int32)])
    src2 = src2.reshape(EPAD // CHUNK, CHUNK)
    dst2 = dst2.reshape(EPAD // CHUNK, CHUNK)

    cnt_pk = _sc_count(dst2)
    cnt = cnt_pk.reshape(NC, NPAD)[:, :N, None]
    p1 = _sc_agg(x, src2, dst2)
    h = _tc_layer(p1, cnt, x, W1l.T, b1l, W1r.T, relu=True)
    p2 = _sc_agg(h, src2, dst2)
    return _tc_layer(p2, cnt, h, W2l.T, b2l, W2r.T, relu=False)


# speedup vs baseline: 3.8762x; 1.0382x over previous
import dataclasses

import jax
import jax.numpy as jnp
from jax import lax
from jax.experimental import pallas as pl
from jax.experimental.pallas import tpu as pltpu
from jax.experimental.pallas import tpu_sc as plsc

N = 10000
D = 128
E = 320000
NC = 2
NS = 16
L = 16
CHUNK = 128
BLK = 8
NBLK = 10
NCH = BLK * NBLK
EPW = NCH * CHUNK
EPAD = NC * NS * EPW
NPAD = 10240
RPW = NPAD // NS


def _compiler_params():
    cp = pltpu.CompilerParams()
    if "needs_layout_passes" in pltpu.CompilerParams.__dataclass_fields__:
        cp = dataclasses.replace(cp, needs_layout_passes=False)
    return cp


def _sc_agg(x, src2, dst2):
    mesh = plsc.VectorSubcoreMesh(core_axis_name="c", subcore_axis_name="s")

    def body(x_hbm, src_hbm, dst_hbm, p_hbm, src8, dst8, rows0, rows1, agg,
             sem0, sem1):
        c = lax.axis_index("c")
        s = lax.axis_index("s")

        zero = jnp.zeros((16,), jnp.float32)

        @pl.loop(0, CHUNK)
        def _(i):
            for j in range(D // 16):
                rows0[i, pl.ds(j * 16, 16)] = zero

        for k in range(RPW // CHUNK):
            r0 = s * RPW + k * CHUNK
            pltpu.sync_copy(rows0, agg.at[pl.ds(r0, CHUNK)])
        rem = RPW % CHUNK
        if rem:
            r0 = s * RPW + (RPW // CHUNK) * CHUNK
            pltpu.sync_copy(rows0.at[pl.ds(0, rem)], agg.at[pl.ds(r0, rem)])
        plsc.subcore_barrier()

        wrow = (c * NS + s) * NCH
        rows = (rows0, rows1)
        sems = (sem0, sem1)

        @pl.loop(0, NBLK)
        def _(b):
            blk = wrow + b * BLK
            pltpu.sync_copy(src_hbm.at[pl.ds(blk, BLK)], src8)
            pltpu.sync_copy(dst_hbm.at[pl.ds(blk, BLK)], dst8)
            pltpu.async_copy(x_hbm.at[src8.at[0]], rows0, sem0)
            for j in range(BLK):
                if j + 1 < BLK:
                    pltpu.async_copy(x_hbm.at[src8.at[j + 1]],
                                     rows[(j + 1) % 2], sems[(j + 1) % 2])
                pltpu.make_async_copy(x_hbm.at[src8.at[j]], rows[j % 2],
                                      sems[j % 2]).wait()
                pltpu.sync_copy(rows[j % 2], agg.at[dst8.at[j]], add=True)

        plsc.subcore_barrier()

        r0 = s * RPW
        pltpu.sync_copy(agg.at[pl.ds(r0, RPW)],
                        p_hbm.at[pl.ds(c * NPAD + r0, RPW)])

    k = pl.kernel(
        body, mesh=mesh,
        out_type=jax.ShapeDtypeStruct((NC * NPAD, D), jnp.float32),
        scratch_types=[
            pltpu.VMEM((BLK, CHUNK), jnp.int32),
            pltpu.VMEM((BLK, CHUNK), jnp.int32),
            pltpu.VMEM((CHUNK, D), jnp.float32),
            pltpu.VMEM((CHUNK, D), jnp.float32),
            pltpu.VMEM_SHARED((NPAD, D), jnp.float32),
            pltpu.SemaphoreType.DMA,
            pltpu.SemaphoreType.DMA,
        ],
        compiler_params=_compiler_params())
    return k(x, src2, dst2)


def _sc_count(dst2):
    mesh = plsc.VectorSubcoreMesh(core_axis_name="c", subcore_axis_name="s")

    def body(dst_hbm, cnt_hbm, dst8, cntloc, redbuf, cpk, stage):
        c = lax.axis_index("c")
        s = lax.axis_index("s")

        @pl.loop(0, NPAD // L)
        def _(i):
            cntloc[pl.ds(i * L, L)] = jnp.zeros((L,), jnp.float32)

        wrow = (c * NS + s) * NCH
        ones = jnp.zeros((L,), jnp.float32) + 1.0

        @pl.loop(0, NBLK)
        def _(b):
            pltpu.sync_copy(dst_hbm.at[pl.ds(wrow + b * BLK, BLK)], dst8)
            for j in range(BLK):
                for k in range(CHUNK // L):
                    plsc.addupdate_scatter(
                        cntloc, [dst8[j, pl.ds(k * L, L)]], ones)

        pltpu.sync_copy(cntloc, stage.at[s])
        plsc.subcore_barrier()

        @pl.when(s < NPAD // 1024)
        def _():
            for r in range(8):
                pltpu.sync_copy(
                    stage.at[:, pl.ds(s * 1024 + r * 128, 128)], redbuf)

                for g in range(128 // L):
                    acc = redbuf[0, pl.ds(g * L, L)]
                    for t in range(1, NS):
                        acc += redbuf[t, pl.ds(g * L, L)]
                    cpk[r, pl.ds(g * L, L)] = acc

            pltpu.sync_copy(
                cpk, cnt_hbm.at[pl.ds(c * (NPAD // 128) + s * 8, 8)])

    k = pl.kernel(
        body, mesh=mesh,
        out_type=jax.ShapeDtypeStruct((NC * NPAD // 128, 128), jnp.float32),
        scratch_types=[
            pltpu.VMEM((BLK, CHUNK), jnp.int32),
            pltpu.VMEM((NPAD,), jnp.float32),
            pltpu.VMEM((NS, 128), jnp.float32),
            pltpu.VMEM((8, 128), jnp.float32),
            pltpu.VMEM_SHARED((NS, NPAD), jnp.float32),
        ],
        compiler_params=_compiler_params())
    return k(dst2)


def _tc_layer(p, cnt, x, WlT, bl, WrT, relu):
    def body(p0_ref, p1_ref, c0_ref, c1_ref, x_ref, wl_ref, bl_ref, wr_ref,
             o_ref):
        cntv = c0_ref[...] + c1_ref[...]
        mean = (p0_ref[...] + p1_ref[...]) / jnp.maximum(cntv, 1.0)
        acc = jnp.dot(mean, wl_ref[...], precision=lax.Precision.HIGHEST,
                      preferred_element_type=jnp.float32)
        acc += jnp.dot(x_ref[...], wr_ref[...],
                       precision=lax.Precision.HIGHEST,
                       preferred_element_type=jnp.float32)
        acc += bl_ref[...]
        o_ref[...] = jnp.maximum(acc, 0.0) if relu else acc

    R = 2000
    row = pl.BlockSpec((R, D), lambda i: (i, 0))
    row1 = pl.BlockSpec((R, 1), lambda i: (i, 0))
    full = pl.BlockSpec((D, D), lambda i: (0, 0))
    return pl.pallas_call(
        body,
        grid=(N // R,),
        in_specs=[row, row, row1, row1, row, full,
                  pl.BlockSpec((1, D), lambda i: (0, 0)), full],
        out_specs=row,
        out_shape=jax.ShapeDtypeStruct((N, D), jnp.float32),
    )(p[:N], p[NPAD:NPAD + N], cnt[0], cnt[1], x, WlT,
      bl.reshape(1, D), WrT)


def kernel(x, edge_index, W1l, b1l, W1r, W2l, b2l, W2r):
    src = edge_index[0].astype(jnp.int32)
    dst = edge_index[1].astype(jnp.int32)
    pad = EPAD - E
    src2 = jnp.concatenate([src, jnp.zeros((pad,), jnp.int32)])
    dst2 = jnp.concatenate([dst, jnp.full((pad,), N, jnp.int32)])
    src2 = src2.reshape(EPAD // CHUNK, CHUNK)
    dst2 = dst2.reshape(EPAD // CHUNK, CHUNK)

    cnt_pk = _sc_count(dst2)
    cnt = cnt_pk.reshape(NC, NPAD)[:, :N, None]
    p1 = _sc_agg(x, src2, dst2)
    h = _tc_layer(p1, cnt, x, W1l.T, b1l, W1r.T, relu=True)
    p2 = _sc_agg(h, src2, dst2)
    return _tc_layer(p2, cnt, h, W2l.T, b2l, W2r.T, relu=False)
